# baseline (speedup 1.0000x reference)
import jax
import jax.numpy as jnp
from jax.experimental import pallas as pl

NUM_GRAPHS = 10
SIZES = [10000, 5000, 2500, 1250]
LAYERS = 3


def _gat(x, src, dst, valid, W, a_s, a_d, b, n):
    loop = jnp.arange(n)
    s = jnp.concatenate([src, loop])
    d = jnp.concatenate([dst, loop])
    v = jnp.concatenate([valid, jnp.ones((n,), dtype=bool)])
    h = x @ W
    e = jax.nn.leaky_relu((h @ a_s)[s] + (h @ a_d)[d], 0.2)
    e = jnp.where(v, e, -1e9)
    m = jax.ops.segment_max(e, d, num_segments=n)
    p = jnp.exp(e - m[d])
    p = jnp.where(v, p, 0.0)
    denom = jax.ops.segment_sum(p, d, num_segments=n)
    alpha = p / (denom[d] + 1e-16)
    out = jax.ops.segment_sum(alpha[:, None] * h[s], d, num_segments=n)
    return out + b


def _pool_edges(src, dst, cluster, n_pool):
    s = cluster[src]
    d = cluster[dst]
    ids = s * n_pool + d
    ids = jnp.where(s == d, -1, ids)
    uid = jnp.unique(ids, size=ids.shape[0], fill_value=-1)
    valid = uid >= 0
    uidc = jnp.where(valid, uid, 0)
    return uidc // n_pool, uidc % n_pool, valid


def _bn_pallas(x):
    def body(x_ref, o_ref):
        xv = x_ref[...]
        mu = jnp.mean(xv, axis=0, keepdims=True)
        var = jnp.mean((xv - mu) ** 2, axis=0, keepdims=True)
        o_ref[...] = (xv - mu) / jnp.sqrt(var + 1e-5)

    return pl.pallas_call(
        body,
        out_shape=jax.ShapeDtypeStruct(x.shape, x.dtype),
    )(x)


def kernel(x, edge_index, W0, as0, ad0, b0, W1, as1, ad1, b1, W2, as2, ad2, b2):
    params = [(W0, as0, ad0, b0), (W1, as1, ad1, b1), (W2, as2, ad2, b2)]
    src = edge_index[0]
    dst = edge_index[1]
    valid = jnp.ones((src.shape[0],), dtype=bool)
    h = x
    for i in range(LAYERS):
        n = SIZES[i]
        n_next = SIZES[i + 1]
        W, a_s, a_d, b = params[i]
        h = jax.nn.relu(_gat(h, src, dst, valid, W, a_s, a_d, b, n))
        cluster = jnp.arange(n) // 2
        h = jax.ops.segment_max(h, cluster, num_segments=n_next)
        if i + 1 < LAYERS:
            src, dst, valid = _pool_edges(src, dst, cluster, n_next)
        h = _bn_pallas(h)
    return h.reshape(NUM_GRAPHS, -1)


# speedup vs baseline: 20.1650x; 20.1650x over previous
import functools

import jax
import jax.numpy as jnp
from jax import lax
from jax.experimental import pallas as pl

G = 10
NPG = 1000
F = 128


def _build_counts(src, dst):
    g = dst // NPG
    sl = src % NPG
    dl = dst % NPG
    a0 = jnp.zeros((G, NPG, NPG), jnp.float32).at[g, dl, sl].add(1.0)
    a1 = jnp.zeros((G, NPG // 2, NPG // 2), jnp.float32).at[g, dl // 2, sl // 2].add(1.0)
    a2 = jnp.zeros((G, NPG // 4, NPG // 4), jnp.float32).at[g, dl // 4, sl // 4].add(1.0)
    return a0, a1, a2


def _layer_body(n, first, n_prev_rows, *refs):
    if first:
        a_ref, h_ref, w_ref, as_ref, ad_ref, b_ref, oh_ref, os1_ref, os2_ref = refs
    else:
        a_ref, h_ref, w_ref, as_ref, ad_ref, b_ref, s1_ref, s2_ref, oh_ref, os1_ref, os2_ref = refs
    h = h_ref[0]
    if not first:
        mu = s1_ref[...] / n_prev_rows
        var = jnp.maximum(s2_ref[...] / n_prev_rows - mu * mu, 0.0)
        h = (h - mu) * lax.rsqrt(var + 1e-5)
    h = jnp.dot(h, w_ref[...], preferred_element_type=jnp.float32, precision=lax.Precision.HIGHEST)
    hd = jnp.dot(h, ad_ref[...], preferred_element_type=jnp.float32, precision=lax.Precision.HIGHEST)
    hs = lax.dot_general(as_ref[...], h, (((1,), (1,)), ((), ())),
                         preferred_element_type=jnp.float32, precision=lax.Precision.HIGHEST)
    e = hd + hs
    e = jnp.where(e >= 0, e, 0.2 * e)
    row = lax.broadcasted_iota(jnp.int32, (n, n), 0)
    col = lax.broadcasted_iota(jnp.int32, (n, n), 1)
    diag = row == col
    a = a_ref[0]
    if first:
        m_mat = a + jnp.where(diag, 1.0, 0.0)
    else:
        m_mat = jnp.where(diag, 1.0, jnp.where(a > 0, 1.0, 0.0))
    en = jnp.where(m_mat > 0, e, -1e30)
    mx = jnp.max(en, axis=1, keepdims=True)
    p = m_mat * jnp.exp(en - mx)
    denom = jnp.sum(p, axis=1, keepdims=True) + 1e-16
    out = jnp.dot(p / denom, h, preferred_element_type=jnp.float32, precision=lax.Precision.HIGHEST) + b_ref[...]
    out = jnp.maximum(out, 0.0)
    pooled = jnp.max(out.reshape(n // 2, 2, F), axis=1)
    oh_ref[0] = pooled

    @pl.when(pl.program_id(0) == 0)
    def _():
        os1_ref[...] = jnp.zeros_like(os1_ref)
        os2_ref[...] = jnp.zeros_like(os2_ref)

    os1_ref[...] += jnp.sum(pooled, axis=0, keepdims=True)
    os2_ref[...] += jnp.sum(pooled * pooled, axis=0, keepdims=True)


def _gat_layer(a, h, w, a_s, a_d, b, sums, n, first, n_prev_rows):
    vec_spec = pl.BlockSpec((1, F), lambda g: (0, 0))
    in_specs = [
        pl.BlockSpec((1, n, n), lambda g: (g, 0, 0)),
        pl.BlockSpec((1, n, F), lambda g: (g, 0, 0)),
        pl.BlockSpec((F, F), lambda g: (0, 0)),
        pl.BlockSpec((1, F), lambda g: (0, 0)),
        pl.BlockSpec((F, 1), lambda g: (0, 0)),
        vec_spec,
    ]
    args = [a, h, w, a_s.reshape(1, F), a_d.reshape(F, 1), b.reshape(1, F)]
    if not first:
        in_specs += [vec_spec, vec_spec]
        args += [sums[0], sums[1]]
    return pl.pallas_call(
        functools.partial(_layer_body, n, first, n_prev_rows),
        grid=(G,),
        in_specs=in_specs,
        out_specs=[
            pl.BlockSpec((1, n // 2, F), lambda g: (g, 0, 0)),
            vec_spec,
            vec_spec,
        ],
        out_shape=[
            jax.ShapeDtypeStruct((G, n // 2, F), jnp.float32),
            jax.ShapeDtypeStruct((1, F), jnp.float32),
            jax.ShapeDtypeStruct((1, F), jnp.float32),
        ],
    )(*args)


def _bn_final_body(h_ref, s1_ref, s2_ref, o_ref):
    h = h_ref[...]
    n_rows = h.shape[0] * h.shape[1]
    mu = s1_ref[...] / n_rows
    var = jnp.maximum(s2_ref[...] / n_rows - mu * mu, 0.0)
    o_ref[...] = (h - mu[None]) * lax.rsqrt(var[None] + 1e-5)


def _bn_final(h, s1, s2):
    return pl.pallas_call(
        _bn_final_body,
        out_shape=jax.ShapeDtypeStruct(h.shape, jnp.float32),
    )(h, s1, s2)


def kernel(x, edge_index, W0, as0, ad0, b0, W1, as1, ad1, b1, W2, as2, ad2, b2):
    a0, a1, a2 = _build_counts(edge_index[0], edge_index[1])
    h = x.reshape(G, NPG, F)
    h, s1, s2 = _gat_layer(a0, h, W0, as0, ad0, b0, None, NPG, True, 0)
    h, s1, s2 = _gat_layer(a1, h, W1, as1, ad1, b1, (s1, s2), NPG // 2, False, G * NPG // 2)
    h, s1, s2 = _gat_layer(a2, h, W2, as2, ad2, b2, (s1, s2), NPG // 4, False, G * NPG // 4)
    out = _bn_final(h, s1, s2)
    return out.reshape(G, -1)


# speedup vs baseline: 100.7901x; 4.9983x over previous
import functools

import jax
import jax.numpy as jnp
from jax import lax
from jax.experimental import pallas as pl
from jax.experimental.pallas import tpu as pltpu
from jax.experimental.pallas import tpu_sc as plsc

G = 10
NPG = 1000
F = 128
EPG = 32000
NC, NS, LANES = 2, 16, 16
NW = NC * NS
R0 = 32
A2W = 256


def _sc_counts(src, dst):
    mesh = plsc.VectorSubcoreMesh(core_axis_name="c", subcore_axis_name="s")
    out_type = [
        jax.ShapeDtypeStruct((G * NPG * NPG,), jnp.float32),
        jax.ShapeDtypeStruct((G * (NPG // 2) * (NPG // 2),), jnp.float32),
        jax.ShapeDtypeStruct((G * (NPG // 4) * A2W,), jnp.float32),
    ]
    scratch = [
        pltpu.VMEM((EPG,), jnp.int32),
        pltpu.VMEM((EPG,), jnp.int32),
        pltpu.VMEM((R0 * NPG,), jnp.float32),
        pltpu.VMEM(((R0 // 2) * (NPG // 2),), jnp.float32),
        pltpu.VMEM(((R0 // 4) * A2W,), jnp.float32),
        pltpu.SemaphoreType.DMA,
        pltpu.SemaphoreType.DMA,
        pltpu.SemaphoreType.DMA,
        pltpu.SemaphoreType.DMA,
        pltpu.SemaphoreType.DMA,
    ]
    UNROLL = 4

    @functools.partial(
        pl.kernel, mesh=mesh, out_type=out_type, scratch_types=scratch,
        compiler_params=pltpu.CompilerParams(needs_layout_passes=False))
    def k(src_hbm, dst_hbm, a0_hbm, a1_hbm, a2_hbm, s_v, d_v, a0_v, a1_v, a2_v,
          sem_s, sem_d, sem0, sem1, sem2):
        w = lax.axis_index("s") * NC + lax.axis_index("c")
        r0 = jnp.minimum(w * R0, NPG - R0)
        ones = jnp.full((LANES,), 1.0, jnp.float32)
        zeros = jnp.zeros((LANES,), jnp.float32)

        def in_copies(g):
            return (pltpu.make_async_copy(src_hbm.at[pl.ds(g * EPG, EPG)], s_v, sem_s),
                    pltpu.make_async_copy(dst_hbm.at[pl.ds(g * EPG, EPG)], d_v, sem_d))

        def out_copies(g):
            return (
                pltpu.make_async_copy(
                    a0_v, a0_hbm.at[pl.ds(g * NPG * NPG + r0 * NPG, R0 * NPG)], sem0),
                pltpu.make_async_copy(
                    a1_v,
                    a1_hbm.at[pl.ds(g * (NPG // 2) * (NPG // 2) + (r0 // 4) * NPG,
                                    (R0 // 2) * (NPG // 2))], sem1),
                pltpu.make_async_copy(
                    a2_v,
                    a2_hbm.at[pl.ds(g * (NPG // 4) * A2W + (r0 // 4) * A2W,
                                    (R0 // 4) * A2W)], sem2),
            )

        def zero_loop(buf, n):
            def zb(i, c):
                buf[pl.ds(i * LANES, LANES)] = zeros
                return c

            lax.fori_loop(0, n // LANES, zb, 0)

        for c in in_copies(0):
            c.start()

        def graph_body(g, carry):
            @pl.when(g > 0)
            def _():
                for c in out_copies(g - 1):
                    c.wait()

            zero_loop(a0_v, R0 * NPG)
            zero_loop(a1_v, (R0 // 2) * (NPG // 2))
            zero_loop(a2_v, (R0 // 4) * A2W)
            for c in in_copies(g):
                c.wait()
            base = g * NPG
            b0 = base + r0

            def chunk(i, c):
                for j in range(UNROLL):
                    off = i * (LANES * UNROLL) + j * LANES
                    s = s_v[pl.ds(off, LANES)]
                    d = d_v[pl.ds(off, LANES)]
                    sl = s - base
                    dr = d - b0
                    mask = (dr >= 0) & (dr < R0)
                    plsc.addupdate_scatter(a0_v, [dr * NPG + sl], ones, mask=mask)
                    dl1 = lax.shift_right_logical(d - base, 1)
                    sl1 = lax.shift_right_logical(sl, 1)
                    dr1 = dl1 - lax.shift_right_logical(r0, 1)
                    plsc.addupdate_scatter(a1_v, [dr1 * (NPG // 2) + sl1], ones, mask=mask)
                    dl2 = lax.shift_right_logical(d - base, 2)
                    sl2 = lax.shift_right_logical(sl, 2)
                    dr2 = dl2 - lax.shift_right_logical(r0, 2)
                    plsc.addupdate_scatter(a2_v, [dr2 * A2W + sl2], ones, mask=mask)
                return c

            lax.fori_loop(0, EPG // (LANES * UNROLL), chunk, 0)
            for c in out_copies(g):
                c.start()

            @pl.when(g < G - 1)
            def _():
                for c in in_copies(g + 1):
                    c.start()

            return carry

        lax.fori_loop(0, G, graph_body, 0)
        for c in out_copies(G - 1):
            c.wait()

    a0, a1, a2 = k(src, dst)
    return (a0.reshape(G, NPG, NPG),
            a1.reshape(G, NPG // 2, NPG // 2),
            a2.reshape(G, NPG // 4, A2W))


def _layer_body(n, first, n_prev_rows, *refs):
    if first:
        a_ref, h_ref, w_ref, as_ref, ad_ref, b_ref, oh_ref, os1_ref, os2_ref = refs
    else:
        a_ref, h_ref, w_ref, as_ref, ad_ref, b_ref, s1_ref, s2_ref, oh_ref, os1_ref, os2_ref = refs
    h = h_ref[0]
    if not first:
        mu = s1_ref[...] / n_prev_rows
        var = jnp.maximum(s2_ref[...] / n_prev_rows - mu * mu, 0.0)
        h = (h - mu) * lax.rsqrt(var + 1e-5)
    h = jnp.dot(h, w_ref[...], preferred_element_type=jnp.float32, precision=lax.Precision.DEFAULT)
    hd = jnp.dot(h, ad_ref[...], preferred_element_type=jnp.float32, precision=lax.Precision.DEFAULT)
    hs = lax.dot_general(as_ref[...], h, (((1,), (1,)), ((), ())),
                         preferred_element_type=jnp.float32, precision=lax.Precision.DEFAULT)
    e = hd + hs
    e = jnp.where(e >= 0, e, 0.2 * e)
    row = lax.broadcasted_iota(jnp.int32, (n, n), 0)
    col = lax.broadcasted_iota(jnp.int32, (n, n), 1)
    diag = row == col
    a = a_ref[0][:, :n]
    if first:
        m_mat = a + jnp.where(diag, 1.0, 0.0)
    else:
        m_mat = jnp.where(diag, 1.0, jnp.where(a > 0, 1.0, 0.0))
    en = jnp.where(m_mat > 0, e, -1e30)
    mx = jnp.max(en, axis=1, keepdims=True)
    p = m_mat * jnp.exp(en - mx)
    denom = jnp.sum(p, axis=1, keepdims=True) + 1e-16
    out = jnp.dot(p / denom, h, preferred_element_type=jnp.float32, precision=lax.Precision.HIGHEST) + b_ref[...]
    out = jnp.maximum(out, 0.0)
    pooled = jnp.max(out.reshape(n // 2, 2, F), axis=1)
    oh_ref[0] = pooled

    @pl.when(pl.program_id(0) == 0)
    def _():
        os1_ref[...] = jnp.zeros_like(os1_ref)
        os2_ref[...] = jnp.zeros_like(os2_ref)

    os1_ref[...] += jnp.sum(pooled, axis=0, keepdims=True)
    os2_ref[...] += jnp.sum(pooled * pooled, axis=0, keepdims=True)


def _gat_layer(a, h, w, a_s, a_d, b, sums, n, first, n_prev_rows):
    vec_spec = pl.BlockSpec((1, F), lambda g: (0, 0))
    acols = a.shape[2]
    in_specs = [
        pl.BlockSpec((1, n, acols), lambda g: (g, 0, 0)),
        pl.BlockSpec((1, n, F), lambda g: (g, 0, 0)),
        pl.BlockSpec((F, F), lambda g: (0, 0)),
        pl.BlockSpec((1, F), lambda g: (0, 0)),
        pl.BlockSpec((F, 1), lambda g: (0, 0)),
        vec_spec,
    ]
    args = [a, h, w, a_s.reshape(1, F), a_d.reshape(F, 1), b.reshape(1, F)]
    if not first:
        in_specs += [vec_spec, vec_spec]
        args += [sums[0], sums[1]]
    return pl.pallas_call(
        functools.partial(_layer_body, n, first, n_prev_rows),
        grid=(G,),
        in_specs=in_specs,
        out_specs=[
            pl.BlockSpec((1, n // 2, F), lambda g: (g, 0, 0)),
            vec_spec,
            vec_spec,
        ],
        out_shape=[
            jax.ShapeDtypeStruct((G, n // 2, F), jnp.float32),
            jax.ShapeDtypeStruct((1, F), jnp.float32),
            jax.ShapeDtypeStruct((1, F), jnp.float32),
        ],
    )(*args)


def _bn_final_body(h_ref, s1_ref, s2_ref, o_ref):
    h = h_ref[...]
    n_rows = h.shape[0] * h.shape[1]
    mu = s1_ref[...] / n_rows
    var = jnp.maximum(s2_ref[...] / n_rows - mu * mu, 0.0)
    o_ref[...] = (h - mu[None]) * lax.rsqrt(var[None] + 1e-5)


def _bn_final(h, s1, s2):
    return pl.pallas_call(
        _bn_final_body,
        out_shape=jax.ShapeDtypeStruct(h.shape, jnp.float32),
    )(h, s1, s2)


def kernel(x, edge_index, W0, as0, ad0, b0, W1, as1, ad1, b1, W2, as2, ad2, b2):
    a0, a1, a2 = _sc_counts(edge_index[0], edge_index[1])
    h = x.reshape(G, NPG, F)
    h, s1, s2 = _gat_layer(a0, h, W0, as0, ad0, b0, None, NPG, True, 0)
    h, s1, s2 = _gat_layer(a1, h, W1, as1, ad1, b1, (s1, s2), NPG // 2, False, G * NPG // 2)
    h, s1, s2 = _gat_layer(a2, h, W2, as2, ad2, b2, (s1, s2), NPG // 4, False, G * NPG // 4)
    out = _bn_final(h, s1, s2)
    return out.reshape(G, -1)


# speedup vs baseline: 116.5141x; 1.1560x over previous
import functools

import jax
import jax.numpy as jnp
from jax import lax
from jax.experimental import pallas as pl
from jax.experimental.pallas import tpu as pltpu
from jax.experimental.pallas import tpu_sc as plsc

G = 10
NPG = 1000
F = 128
EPG = 32000
E_TOT = G * EPG
NC, NS, LANES = 2, 16, 16
NW = NC * NS
R0 = 32
A2W = 256


def _pack_edges_body(s_ref, d_ref, o_ref):
    s = s_ref[...]
    d = d_ref[...]
    gbase = (d // NPG) * NPG
    o_ref[...] = lax.shift_left(d - gbase, 10) | (s - gbase)


def _pack_edges(edge_index):
    ei = edge_index.reshape(2, 1, E_TOT)
    return pl.pallas_call(
        _pack_edges_body,
        out_shape=jax.ShapeDtypeStruct((1, E_TOT), jnp.int32),
    )(ei[0], ei[1]).reshape(E_TOT)


def _sc_counts(packed):
    mesh = plsc.VectorSubcoreMesh(core_axis_name="c", subcore_axis_name="s")
    out_type = [
        jax.ShapeDtypeStruct((G * NPG * NPG,), jnp.float32),
        jax.ShapeDtypeStruct((G * (NPG // 2) * (NPG // 2),), jnp.float32),
        jax.ShapeDtypeStruct((G * (NPG // 4) * A2W,), jnp.float32),
    ]
    scratch = [
        pltpu.VMEM((EPG,), jnp.int32),
        pltpu.VMEM((R0 * NPG,), jnp.float32),
        pltpu.VMEM(((R0 // 2) * (NPG // 2),), jnp.float32),
        pltpu.VMEM(((R0 // 4) * A2W,), jnp.float32),
        pltpu.SemaphoreType.DMA,
        pltpu.SemaphoreType.DMA,
        pltpu.SemaphoreType.DMA,
        pltpu.SemaphoreType.DMA,
    ]
    UNROLL = 8

    @functools.partial(
        pl.kernel, mesh=mesh, out_type=out_type, scratch_types=scratch,
        compiler_params=pltpu.CompilerParams(needs_layout_passes=False))
    def k(p_hbm, a0_hbm, a1_hbm, a2_hbm, p_v, a0_v, a1_v, a2_v,
          sem_s, sem0, sem1, sem2):
        w = lax.axis_index("s") * NC + lax.axis_index("c")
        r0 = jnp.minimum(w * R0, NPG - R0)
        ones = jnp.full((LANES,), 1.0, jnp.float32)
        zeros = jnp.zeros((LANES,), jnp.float32)
        r0u = jnp.uint32(R0)

        def in_copies(g):
            return (pltpu.make_async_copy(p_hbm.at[pl.ds(g * EPG, EPG)], p_v, sem_s),)

        def out_copies(g):
            return (
                pltpu.make_async_copy(
                    a0_v, a0_hbm.at[pl.ds(g * NPG * NPG + r0 * NPG, R0 * NPG)], sem0),
                pltpu.make_async_copy(
                    a1_v,
                    a1_hbm.at[pl.ds(g * (NPG // 2) * (NPG // 2) + (r0 // 4) * NPG,
                                    (R0 // 2) * (NPG // 2))], sem1),
                pltpu.make_async_copy(
                    a2_v,
                    a2_hbm.at[pl.ds(g * (NPG // 4) * A2W + (r0 // 4) * A2W,
                                    (R0 // 4) * A2W)], sem2),
            )

        def zero_loop(buf, n):
            zu = 8 if n % (8 * LANES) == 0 else 1

            def zb(i, c):
                for j in range(zu):
                    buf[pl.ds(i * (zu * LANES) + j * LANES, LANES)] = zeros
                return c

            lax.fori_loop(0, n // (zu * LANES), zb, 0)

        for c in in_copies(0):
            c.start()

        def graph_body(g, carry):
            @pl.when(g > 0)
            def _():
                for c in out_copies(g - 1):
                    c.wait()

            zero_loop(a0_v, R0 * NPG)
            zero_loop(a1_v, (R0 // 2) * (NPG // 2))
            zero_loop(a2_v, (R0 // 4) * A2W)
            for c in in_copies(g):
                c.wait()

            def chunk(i, c):
                for j in range(UNROLL):
                    off = i * (LANES * UNROLL) + j * LANES
                    p = p_v[pl.ds(off, LANES)]
                    sl = p & 1023
                    dr = lax.shift_right_logical(p, 10) - r0
                    mask = plsc.bitcast(dr, jnp.uint32) < r0u
                    plsc.addupdate_scatter(a0_v, [dr * NPG + sl], ones, mask=mask)
                    dr1 = lax.shift_right_logical(dr, 1)
                    sl1 = lax.shift_right_logical(sl, 1)
                    plsc.addupdate_scatter(a1_v, [dr1 * (NPG // 2) + sl1], ones, mask=mask)
                    dr2 = lax.shift_right_logical(dr, 2)
                    sl2 = lax.shift_right_logical(sl, 2)
                    plsc.addupdate_scatter(a2_v, [dr2 * A2W + sl2], ones, mask=mask)
                return c

            lax.fori_loop(0, EPG // (LANES * UNROLL), chunk, 0)
            for c in out_copies(g):
                c.start()

            @pl.when(g < G - 1)
            def _():
                for c in in_copies(g + 1):
                    c.start()

            return carry

        lax.fori_loop(0, G, graph_body, 0)
        for c in out_copies(G - 1):
            c.wait()

    a0, a1, a2 = k(packed)
    return (a0.reshape(G, NPG, NPG),
            a1.reshape(G, NPG // 2, NPG // 2),
            a2.reshape(G, NPG // 4, A2W))


def _layer_body(n, first, n_prev_rows, *refs):
    if first:
        a_ref, h_ref, w_ref, as_ref, ad_ref, b_ref, oh_ref, os1_ref, os2_ref = refs
    else:
        a_ref, h_ref, w_ref, as_ref, ad_ref, b_ref, s1_ref, s2_ref, oh_ref, os1_ref, os2_ref = refs
    h = h_ref[0]
    if not first:
        mu = s1_ref[...] / n_prev_rows
        var = jnp.maximum(s2_ref[...] / n_prev_rows - mu * mu, 0.0)
        h = (h - mu) * lax.rsqrt(var + 1e-5)
    h = jnp.dot(h, w_ref[...], preferred_element_type=jnp.float32, precision=lax.Precision.DEFAULT)
    hd = jnp.dot(h, ad_ref[...], preferred_element_type=jnp.float32, precision=lax.Precision.DEFAULT)
    hs = lax.dot_general(as_ref[...], h, (((1,), (1,)), ((), ())),
                         preferred_element_type=jnp.float32, precision=lax.Precision.DEFAULT)
    e = hd + hs
    e = jnp.where(e >= 0, e, 0.2 * e)
    row = lax.broadcasted_iota(jnp.int32, (n, n), 0)
    col = lax.broadcasted_iota(jnp.int32, (n, n), 1)
    diag = row == col
    a = a_ref[0][:, :n]
    if first:
        m_mat = a + jnp.where(diag, 1.0, 0.0)
    else:
        m_mat = jnp.where(diag, 1.0, jnp.where(a > 0, 1.0, 0.0))
    en = jnp.where(m_mat > 0, e, -1e30)
    mx = jnp.max(en, axis=1, keepdims=True)
    p = m_mat * jnp.exp(en - mx)
    denom = jnp.sum(p, axis=1, keepdims=True) + 1e-16
    out = jnp.dot(p / denom, h, preferred_element_type=jnp.float32, precision=lax.Precision.HIGHEST) + b_ref[...]
    out = jnp.maximum(out, 0.0)
    pooled = jnp.max(out.reshape(n // 2, 2, F), axis=1)
    oh_ref[0] = pooled

    @pl.when(pl.program_id(0) == 0)
    def _():
        os1_ref[...] = jnp.zeros_like(os1_ref)
        os2_ref[...] = jnp.zeros_like(os2_ref)

    os1_ref[...] += jnp.sum(pooled, axis=0, keepdims=True)
    os2_ref[...] += jnp.sum(pooled * pooled, axis=0, keepdims=True)


def _gat_layer(a, h, w, a_s, a_d, b, sums, n, first, n_prev_rows):
    vec_spec = pl.BlockSpec((1, F), lambda g: (0, 0))
    acols = a.shape[2]
    in_specs = [
        pl.BlockSpec((1, n, acols), lambda g: (g, 0, 0)),
        pl.BlockSpec((1, n, F), lambda g: (g, 0, 0)),
        pl.BlockSpec((F, F), lambda g: (0, 0)),
        pl.BlockSpec((1, F), lambda g: (0, 0)),
        pl.BlockSpec((F, 1), lambda g: (0, 0)),
        vec_spec,
    ]
    args = [a, h, w, a_s.reshape(1, F), a_d.reshape(F, 1), b.reshape(1, F)]
    if not first:
        in_specs += [vec_spec, vec_spec]
        args += [sums[0], sums[1]]
    return pl.pallas_call(
        functools.partial(_layer_body, n, first, n_prev_rows),
        grid=(G,),
        in_specs=in_specs,
        out_specs=[
            pl.BlockSpec((1, n // 2, F), lambda g: (g, 0, 0)),
            vec_spec,
            vec_spec,
        ],
        out_shape=[
            jax.ShapeDtypeStruct((G, n // 2, F), jnp.float32),
            jax.ShapeDtypeStruct((1, F), jnp.float32),
            jax.ShapeDtypeStruct((1, F), jnp.float32),
        ],
    )(*args)


def _bn_final_body(h_ref, s1_ref, s2_ref, o_ref):
    h = h_ref[...]
    n_rows = h.shape[0] * h.shape[1]
    mu = s1_ref[...] / n_rows
    var = jnp.maximum(s2_ref[...] / n_rows - mu * mu, 0.0)
    o_ref[...] = (h - mu[None]) * lax.rsqrt(var[None] + 1e-5)


def _bn_final(h, s1, s2):
    return pl.pallas_call(
        _bn_final_body,
        out_shape=jax.ShapeDtypeStruct(h.shape, jnp.float32),
    )(h, s1, s2)


def kernel(x, edge_index, W0, as0, ad0, b0, W1, as1, ad1, b1, W2, as2, ad2, b2):
    a0, a1, a2 = _sc_counts(_pack_edges(edge_index))
    h = x.reshape(G, NPG, F)
    h, s1, s2 = _gat_layer(a0, h, W0, as0, ad0, b0, None, NPG, True, 0)
    h, s1, s2 = _gat_layer(a1, h, W1, as1, ad1, b1, (s1, s2), NPG // 2, False, G * NPG // 2)
    h, s1, s2 = _gat_layer(a2, h, W2, as2, ad2, b2, (s1, s2), NPG // 4, False, G * NPG // 4)
    out = _bn_final(h, s1, s2)
    return out.reshape(G, -1)
